# baseline (speedup 1.0000x reference)
import jax
import jax.numpy as jnp
from jax import lax
from jax.experimental import pallas as pl
from jax.experimental.pallas import tpu as pltpu
from jax.experimental.pallas import tpu_sc as plsc

_BATCH = 1024
_SEQ = 200
_D = 64
_DP = 128
_NTOK = _BATCH * _SEQ
_VOCAB = 1000000
_NBLK = (_VOCAB + 127) // 128
_NW = 32
_XROWS = _NTOK // _DP
_OUTROWS = _NTOK + _DP
_MCAP = 8192
_OCAP = 1248
_QCAP = 768
_WTOK = 256
_TAIL0 = (_VOCAB // 128) * 128


def _iota16():
    return lax.broadcasted_iota(jnp.int32, (16,), 0)


def _append(dst_refs, idx_prefix, vals, mask, offm1, cap):
    pc = plsc.cumsum(mask.astype(jnp.int32))
    pos = jnp.minimum(offm1 + pc, cap - 1)
    for ref, v in zip(dst_refs, vals):
        plsc.store_scatter(ref, idx_prefix + [pos], v, mask=mask)
    return offm1 + plsc.all_reduce_population_count(mask)


def _sc_body(x_hbm, bt_hbm, tail_hbm, pe_hbm, out_hbm,
             xb, mtok, mslot, otok, oslot, qtok, qslot, bb, pe_v, stg,
             xsem, bsem, ssem):
    wid = lax.axis_index("subcore") * 2 + lax.axis_index("core")
    start_blk = (wid * _NBLK) // _NW
    end_blk = ((wid + 1) * _NBLK) // _NW
    tok_lo = start_blk * 128
    tok_hi = end_blk * 128
    nsb = (end_blk - start_blk + 1) // 2
    iota = _iota16()
    neg1 = jnp.full((16,), -1, jnp.int32)

    pltpu.sync_copy(pe_hbm, pe_v)

    def x_dma(ch):
        return pltpu.make_async_copy(
            x_hbm.at[pl.ds(ch * 8, 8)], xb.at[ch % 2], xsem.at[ch % 2])

    x_dma(0).start()
    x_dma(1).start()

    def a_chunk(ch, carry):
        offm1, _ = carry
        x_dma(ch).wait()
        buf = ch % 2

        def a_row(r, carry_r):
            offm1_r = carry_r
            for c in range(8):
                xv = xb[buf, r, pl.ds(c * 16, 16)]
                m = (xv >= tok_lo) & (xv < tok_hi)
                slotv = ch * 1024 + r * 128 + c * 16 + iota
                offm1_r = _append([mtok, mslot], [], [xv, slotv], m,
                                  offm1_r, _MCAP)
            return offm1_r

        offm1 = lax.fori_loop(0, 8, a_row, offm1)

        @pl.when(ch + 2 < _XROWS // 8)
        def _():
            x_dma(ch + 2).start()

        return offm1, 0

    offm1, _ = lax.fori_loop(0, _XROWS // 8, a_chunk, (neg1, 0))
    cnt = jnp.minimum(offm1[0] + 1, _MCAP)

    osb = [(j * nsb) // 8 for j in range(9)]
    ocnts = []
    nv_m = (cnt + 15) // 16
    for j in range(8):
        o_lo = (start_blk + 2 * osb[j]) * 128
        o_hi = jnp.minimum((start_blk + 2 * osb[j + 1]) * 128, tok_hi)
        jv = jnp.full((16,), j, jnp.int32)

        def b_step(i, carry_b, o_lo=o_lo, o_hi=o_hi, jv=jv, cnt=cnt):
            offm1_b = carry_b
            tv = mtok[pl.ds(i * 16, 16)]
            sv = mslot[pl.ds(i * 16, 16)]
            m = (tv >= o_lo) & (tv < o_hi) & (i * 16 + iota < cnt)
            return _append([otok, oslot], [jv], [tv, sv], m, offm1_b, _OCAP)

        offm1_b = lax.fori_loop(0, nv_m, b_step, neg1)
        ocnts.append(jnp.minimum(offm1_b[0] + 1, _OCAP))

    def b_dma(k):
        t0 = (start_blk + 2 * k) * 128
        buf = k % 2
        full = t0 + _WTOK <= _VOCAB
        return t0, buf, full

    def b_dma_start(k):
        t0, buf, full = b_dma(k)

        @pl.when(full)
        def _():
            pltpu.make_async_copy(
                bt_hbm.at[:, pl.ds(t0, _WTOK)], bb.at[buf], bsem.at[buf]
            ).start()

        @pl.when(jnp.logical_not(full))
        def _():
            pltpu.make_async_copy(
                tail_hbm, bb.at[buf, :, pl.ds(0, 128)], bsem.at[buf]
            ).start()

    def b_dma_wait(k):
        t0, buf, full = b_dma(k)

        @pl.when(full)
        def _():
            pltpu.make_async_copy(
                bt_hbm.at[:, pl.ds(t0, _WTOK)], bb.at[buf], bsem.at[buf]
            ).wait()

        @pl.when(jnp.logical_not(full))
        def _():
            pltpu.make_async_copy(
                tail_hbm, bb.at[buf, :, pl.ds(0, 128)], bsem.at[buf]
            ).wait()

    b_dma_start(0)

    for j in range(8):
        ocnt = ocnts[j]
        nv_o = (ocnt + 15) // 16

        def c_window(k, carry_w, j=j, ocnt=ocnt, nv_o=nv_o):
            k_g = osb[j] + k
            t0 = (start_blk + 2 * k_g) * 128
            buf = k_g % 2
            b_dma_wait(k_g)

            @pl.when(k_g + 1 < nsb)
            def _():
                b_dma_start(k_g + 1)

            def q_step(i, carry_q):
                offm1_q = carry_q
                tv = otok[j, pl.ds(i * 16, 16)]
                sv = oslot[j, pl.ds(i * 16, 16)]
                m = (tv >= t0) & (tv < t0 + _WTOK) & (i * 16 + iota < ocnt)
                return _append([qtok, qslot], [], [tv, sv], m, offm1_q, _QCAP)

            offm1_q = lax.fori_loop(0, nv_o, q_step, neg1)
            qcnt = jnp.minimum(offm1_q[0] + 1, _QCAP)
            tail_pos = jnp.minimum(qcnt + iota, _QCAP + 15)
            plsc.store_scatter(qtok, [tail_pos], jnp.full((16,), 0, jnp.int32) + t0)
            plsc.store_scatter(qslot, [tail_pos], jnp.full((16,), _NTOK, jnp.int32))

            ng = (qcnt + 15) // 16

            def c_group(g, carry_g):
                sbuf = g % 2

                @pl.when(g >= 2)
                def _():
                    pltpu.make_async_copy(
                        stg.at[sbuf], out_hbm.at[iota + _NTOK], ssem.at[sbuf]
                    ).wait()

                tv = qtok[pl.ds(g * 16, 16)]
                sv = qslot[pl.ds(g * 16, 16)]
                toff = tv - t0
                srow = lax.rem(sv, jnp.full((16,), _SEQ, jnp.int32))
                bufv = jnp.full((16,), 0, jnp.int32) + buf
                sbufv = jnp.full((16,), 0, jnp.int32) + sbuf

                def c_feat(f, carry_f):
                    fv = jnp.full((16,), 0, jnp.int32) + f
                    tabv = plsc.load_gather(bb, [bufv, fv, toff])
                    pev = plsc.load_gather(pe_v, [srow, fv])
                    plsc.store_scatter(stg, [sbufv, iota, fv], tabv + pev)
                    return carry_f

                lax.fori_loop(0, _D, c_feat, 0)
                pltpu.make_async_copy(
                    stg.at[sbuf], out_hbm.at[sv], ssem.at[sbuf]).start()
                return carry_g

            lax.fori_loop(0, ng, c_group, 0)

            @pl.when(ng >= 1)
            def _():
                pltpu.make_async_copy(
                    stg.at[(ng - 1) % 2], out_hbm.at[iota + _NTOK],
                    ssem.at[(ng - 1) % 2]).wait()

            @pl.when(ng >= 2)
            def _():
                pltpu.make_async_copy(
                    stg.at[(ng - 2) % 2], out_hbm.at[iota + _NTOK],
                    ssem.at[(ng - 2) % 2]).wait()

            return carry_w

        lax.fori_loop(0, osb[j + 1] - osb[j], c_window, 0)


@jax.jit
def _positional_embedding_stream(xf, bt, tail_pad, pe_pad):
    mesh = plsc.VectorSubcoreMesh(
        core_axis_name="core", subcore_axis_name="subcore")
    kern = pl.kernel(
        _sc_body,
        out_type=jax.ShapeDtypeStruct((_OUTROWS, _DP), jnp.float32),
        mesh=mesh,
        compiler_params=pltpu.CompilerParams(
            use_tc_tiling_on_sc=True, needs_layout_passes=False),
        scratch_types=[
            pltpu.VMEM((2, 8, 128), jnp.int32),
            pltpu.VMEM((_MCAP,), jnp.int32),
            pltpu.VMEM((_MCAP,), jnp.int32),
            pltpu.VMEM((8, _OCAP), jnp.int32),
            pltpu.VMEM((8, _OCAP), jnp.int32),
            pltpu.VMEM((_QCAP + 16,), jnp.int32),
            pltpu.VMEM((_QCAP + 16,), jnp.int32),
            pltpu.VMEM((2, _D, _WTOK), jnp.float32),
            pltpu.VMEM((_SEQ, _DP), jnp.float32),
            pltpu.VMEM((2, 16, _DP), jnp.float32),
            pltpu.SemaphoreType.DMA((2,)),
            pltpu.SemaphoreType.DMA((2,)),
            pltpu.SemaphoreType.DMA((2,)),
        ],
    )
    return kern(xf, bt, tail_pad, pe_pad)


def kernel(X, table, pe):
    seq_len = X.shape[-1]
    pe_tile = pe[0, :seq_len, :]
    pe_pad = jnp.pad(pe_tile, ((0, 0), (0, _DP - _D)))
    xf = X.astype(jnp.int32).reshape(_XROWS, _DP)
    bt = table.T
    tail_pad = jnp.pad(table[_TAIL0:, :].T, ((0, 0), (0, 128 - (_VOCAB - _TAIL0))))
    out = _positional_embedding_stream(xf, bt, tail_pad, pe_pad)
    return out[:_NTOK, :_D].reshape(_BATCH, _SEQ, _D)


# speedup vs baseline: 2.4433x; 2.4433x over previous
import jax
import jax.numpy as jnp
from jax import lax
from jax.experimental import layout as jax_layout
from jax.experimental import pallas as pl
from jax.experimental.pallas import tpu as pltpu
from jax.experimental.pallas import tpu_sc as plsc

_BATCH = 1024
_SEQ = 200
_D = 64
_NC = 2
_NS = 16
_NW = _NC * _NS
_RPW = _BATCH // _NW
_NBUF = 2


def _sc_kernel_body(x_hbm, table_hbm, pe_hbm, out_hbm,
                    idx0_v, idx1_v, rows_v, outb_v, pe_v,
                    isem, gsem, osem):
    wid = lax.axis_index("subcore") * _NC + lax.axis_index("core")
    base = wid * _RPW
    idx_bufs = (idx0_v, idx1_v)

    pltpu.sync_copy(pe_hbm, pe_v)

    def idx_start(b):
        p = b % _NBUF
        pltpu.make_async_copy(
            x_hbm.at[base + b], idx_bufs[p], isem.at[p]).start()

    def idx_wait(b):
        p = b % _NBUF
        pltpu.make_async_copy(
            x_hbm.at[base + b], idx_bufs[p], isem.at[p]).wait()

    def gather_start(b):
        p = b % _NBUF
        pltpu.make_async_copy(
            table_hbm.at[idx_bufs[p]], rows_v.at[p], gsem.at[p]).start()

    def gather_wait(b):
        p = b % _NBUF
        pltpu.make_async_copy(
            table_hbm.at[idx_bufs[p]], rows_v.at[p], gsem.at[p]).wait()

    def out_start(b):
        p = b % _NBUF
        pltpu.make_async_copy(
            outb_v.at[p], out_hbm.at[base + b], osem.at[p]).start()

    def out_wait(b):
        p = b % _NBUF
        pltpu.make_async_copy(
            outb_v.at[p], out_hbm.at[base + b], osem.at[p]).wait()

    for b in range(_NBUF):
        idx_start(b)
        idx_wait(b)
        gather_start(b)

    for b in range(_RPW):
        p = b % _NBUF
        gather_wait(b)
        if b + _NBUF < _RPW:
            idx_start(b + _NBUF)
        if b >= _NBUF:
            out_wait(b - _NBUF)

        @pl.loop(0, _SEQ)
        def _(r, p=p):
            for c in range(0, _D, 16):
                outb_v[p, r, pl.ds(c, 16)] = (
                    rows_v[p, r, pl.ds(c, 16)] + pe_v[r, pl.ds(c, 16)])

        if b + _NBUF < _RPW:
            idx_wait(b + _NBUF)
            gather_start(b + _NBUF)
        out_start(b)

    for b in range(_RPW - _NBUF, _RPW):
        out_wait(b)


def _positional_embedding_sc(x, table, pe_tile):
    mesh = plsc.VectorSubcoreMesh(
        core_axis_name="core", subcore_axis_name="subcore")
    kern = pl.kernel(
        _sc_kernel_body,
        out_type=jax.ShapeDtypeStruct((_BATCH, _SEQ, _D), jnp.float32),
        mesh=mesh,
        compiler_params=pltpu.CompilerParams(use_tc_tiling_on_sc=False),
        scratch_types=[
            pltpu.VMEM((_SEQ,), jnp.int32),
            pltpu.VMEM((_SEQ,), jnp.int32),
            pltpu.VMEM((_NBUF, _SEQ, _D), jnp.float32),
            pltpu.VMEM((_NBUF, _SEQ, _D), jnp.float32),
            pltpu.VMEM((_SEQ, _D), jnp.float32),
            pltpu.SemaphoreType.DMA((_NBUF,)),
            pltpu.SemaphoreType.DMA((_NBUF,)),
            pltpu.SemaphoreType.DMA((_NBUF,)),
        ],
    )
    return kern(x, table, pe_tile)


_positional_embedding_sc_jit = jax.jit(_positional_embedding_sc)


def kernel(X, table, pe):
    seq_len = X.shape[-1]
    pe_tile = pe[0, :seq_len, :]
    return _positional_embedding_sc_jit(X.astype(jnp.int32), table, pe_tile)
